# Initial kernel scaffold; baseline (speedup 1.0000x reference)
import jax
import jax.numpy as jnp
from jax.experimental import pallas as pl


def kernel(tokens, W):
    raise NotImplementedError("write your pallas kernel here")


# speedup vs baseline: 1.2926x; 1.2926x over previous
import functools

import jax
import jax.numpy as jnp
from jax import lax
from jax.experimental import pallas as pl
from jax.experimental.pallas import tpu as pltpu
from jax.experimental.pallas import tpu_sc as plsc

D_EMB = 32
SCALE = float(D_EMB ** 0.5)
NC, NS = 2, 16
NW = NC * NS
GSZ = 128
K = 8
CHUNK = K * GSZ


@functools.partial(jax.jit, static_argnames=())
def _embed(idx2, W):
    n_rows, _ = idx2.shape
    B = n_rows * GSZ
    b_per_w = B // NW
    n_chunks = b_per_w // CHUNK

    mesh = plsc.VectorSubcoreMesh(
        core_axis_name="c", subcore_axis_name="s",
        num_cores=NC, num_subcores=NS)

    @functools.partial(
        pl.kernel,
        out_type=jax.ShapeDtypeStruct((B, D_EMB), jnp.float32),
        mesh=mesh,
        scratch_types=[
            pltpu.VMEM((K, GSZ), jnp.int32),
            pltpu.VMEM((CHUNK, D_EMB), jnp.float32),
            pltpu.SemaphoreType.DMA,
        ],
        compiler_params=pltpu.CompilerParams(use_tc_tiling_on_sc=False),
    )
    def body(idx_hbm, w_hbm, out_hbm, idx_v, rows_v, sem):
        wid = lax.axis_index("s") * NC + lax.axis_index("c")
        base = wid * b_per_w

        @pl.loop(0, n_chunks)
        def chunk(c):
            off = base + c * CHUNK
            rowb = pl.multiple_of(off // GSZ, 8)
            pltpu.sync_copy(idx_hbm.at[pl.ds(rowb, K)], idx_v)
            copies = [
                pltpu.async_copy(
                    w_hbm.at[idx_v.at[j]],
                    rows_v.at[pl.ds(j * GSZ, GSZ)],
                    sem)
                for j in range(K)
            ]
            for cp in copies:
                cp.wait()

            @pl.loop(0, CHUNK)
            def srow(r):
                rows_v[r, pl.ds(0, 16)] = rows_v[r, pl.ds(0, 16)] * SCALE
                rows_v[r, pl.ds(16, 16)] = rows_v[r, pl.ds(16, 16)] * SCALE

            pltpu.sync_copy(rows_v, out_hbm.at[pl.ds(off, CHUNK)])

    return body(idx2, W)


def kernel(tokens, W):
    n_seq, n_tok = tokens.shape
    B = n_seq * n_tok
    idx2 = tokens.reshape(B // GSZ, GSZ).astype(jnp.int32)
    out = _embed(idx2, W)
    return out.reshape(n_seq, n_tok, D_EMB)


# speedup vs baseline: 1.4609x; 1.1302x over previous
import functools

import jax
import jax.numpy as jnp
from jax import lax
from jax.experimental import pallas as pl
from jax.experimental.pallas import tpu as pltpu
from jax.experimental.pallas import tpu_sc as plsc

D_EMB = 32
SCALE = float(D_EMB ** 0.5)
NC, NS = 2, 16
NW = NC * NS
GSZ = 128
K = 5
CHUNK = K * GSZ
NBUF = 4


def _embed(idx3, W):
    n_chunk_total, _, _ = idx3.shape
    n_chunks = n_chunk_total // NW
    n_quads = n_chunks // NBUF
    B = n_chunk_total * CHUNK

    mesh = plsc.VectorSubcoreMesh(
        core_axis_name="c", subcore_axis_name="s",
        num_cores=NC, num_subcores=NS)

    @functools.partial(
        pl.kernel,
        out_type=jax.ShapeDtypeStruct((B, D_EMB), jnp.float32),
        mesh=mesh,
        scratch_types=(
            [pltpu.VMEM((K, GSZ), jnp.int32) for _ in range(NBUF)]
            + [pltpu.VMEM((CHUNK, D_EMB), jnp.float32) for _ in range(NBUF)]
            + [pltpu.SemaphoreType.DMA]
            + [pltpu.SemaphoreType.DMA for _ in range(NBUF)]
            + [pltpu.SemaphoreType.DMA for _ in range(NBUF)]
        ),
        compiler_params=pltpu.CompilerParams(use_tc_tiling_on_sc=False),
    )
    def body(idx_hbm, w_hbm, out_hbm, *refs):
        ivs = refs[0:NBUF]
        bufs = refs[NBUF:2 * NBUF]
        isem = refs[2 * NBUF]
        gsems = refs[2 * NBUF + 1:3 * NBUF + 1]
        osems = refs[3 * NBUF + 1:4 * NBUF + 1]

        wid = lax.axis_index("s") * NC + lax.axis_index("c")
        cbase = wid * n_chunks

        @pl.loop(0, n_quads)
        def quad(q):
            c0 = cbase + q * NBUF
            icps = [
                pltpu.async_copy(idx_hbm.at[c0 + s], ivs[s], isem)
                for s in range(NBUF)
            ]
            for cp in icps:
                cp.wait()
            gcps = []
            for s in range(NBUF):
                @pl.when(q > 0)
                def _(s=s):
                    pltpu.make_async_copy(
                        bufs[s], out_hbm.at[pl.ds(0, CHUNK)], osems[s]
                    ).wait()
                for j in range(K):
                    gcps.append(pltpu.async_copy(
                        w_hbm.at[ivs[s].at[j]],
                        bufs[s].at[pl.ds(j * GSZ, GSZ)],
                        gsems[s]))
            for s in range(NBUF):
                for j in range(K):
                    gcps[s * K + j].wait()

                @plsc.parallel_loop(0, CHUNK, unroll=8)
                def srow(r, s=s):
                    bufs[s][r, pl.ds(0, 16)] = bufs[s][r, pl.ds(0, 16)] * SCALE
                    bufs[s][r, pl.ds(16, 16)] = bufs[s][r, pl.ds(16, 16)] * SCALE

                off = (c0 + s) * CHUNK
                pltpu.async_copy(
                    bufs[s], out_hbm.at[pl.ds(off, CHUNK)], osems[s])

        for s in range(NBUF):
            pltpu.make_async_copy(
                bufs[s], out_hbm.at[pl.ds(0, CHUNK)], osems[s]).wait()

    return body(idx3, W)


def kernel(tokens, W):
    n_seq, n_tok = tokens.shape
    B = n_seq * n_tok
    idx3 = tokens.reshape(B // CHUNK, K, GSZ).astype(jnp.int32)
    out = _embed(idx3, W)
    return out.reshape(n_seq, n_tok, D_EMB)


# speedup vs baseline: 1.7076x; 1.1689x over previous
import functools

import jax
import jax.numpy as jnp
from jax import lax
from jax.experimental import pallas as pl
from jax.experimental.pallas import tpu as pltpu
from jax.experimental.pallas import tpu_sc as plsc

D_EMB = 32
SCALE = float(D_EMB ** 0.5)
NC, NS = 2, 16
NW = NC * NS
D_VOC = 1000000

CW = 512
N_FULL = D_VOC // CW
TAIL = D_VOC - N_FULL * CW
N_PAIRS = 30

SB = 256
KG = SB // 128


def _mesh():
    return plsc.VectorSubcoreMesh(
        core_axis_name="c", subcore_axis_name="s",
        num_cores=NC, num_subcores=NS)


def _relayout(WT, tail16):
    @functools.partial(
        pl.kernel,
        out_type=jax.ShapeDtypeStruct((D_VOC // 4, 128), jnp.float32),
        mesh=_mesh(),
        scratch_types=(
            [pltpu.VMEM((D_EMB, CW), jnp.float32) for _ in range(2)]
            + [pltpu.VMEM((CW // 4, 128), jnp.float32) for _ in range(2)]
            + [pltpu.SemaphoreType.DMA for _ in range(4)]
        ),
        compiler_params=pltpu.CompilerParams(use_tc_tiling_on_sc=True, needs_layout_passes=False),
    )
    def body(wt_hbm, tail_hbm, out_hbm, ia, ib, oa, ob, gsa, gsb, osa, osb):
        wid = lax.axis_index("s") * NC + lax.axis_index("c")
        iota = lax.iota(jnp.int32, 16)

        def fire_in(c, ibuf, gsem):
            v0 = (c * NW + wid) * CW
            return pltpu.async_copy(wt_hbm.at[:, pl.ds(v0, CW)], ibuf, gsem)

        def wait_in(ibuf, gsem):
            pltpu.make_async_copy(
                wt_hbm.at[:, pl.ds(0, CW)], ibuf, gsem).wait()

        def transpose_chunk(ibuf, obuf):
            @plsc.parallel_loop(0, CW // 4, unroll=4)
            def row(r):
                for g in range(8):
                    o, e0 = g // 2, (g % 2) * 16
                    vals = plsc.load_gather(
                        ibuf, [e0 + iota,
                               jnp.full((16,), 4 * r + o, jnp.int32)])
                    obuf[r, pl.ds(16 * g, 16)] = vals * SCALE

        def fire_out(c, obuf, osem):
            r0 = (c * NW + wid) * (CW // 4)
            return pltpu.async_copy(
                obuf, out_hbm.at[pl.ds(r0, CW // 4)], osem)

        def drain_out(osem):
            pltpu.make_async_copy(
                oa, out_hbm.at[pl.ds(0, CW // 4)], osem).wait()

        fire_in(0, ia, gsa)

        @pl.loop(0, N_PAIRS)
        def pair(p):
            fire_in(2 * p + 1, ib, gsb)
            wait_in(ia, gsa)

            @pl.when(p > 0)
            def _():
                drain_out(osa)
            transpose_chunk(ia, oa)
            fire_out(2 * p, oa, osa)
            fire_in(2 * p + 2, ia, gsa)
            wait_in(ib, gsb)

            @pl.when(p > 0)
            def _():
                drain_out(osb)
            transpose_chunk(ib, ob)
            fire_out(2 * p + 1, ob, osb)

        wait_in(ia, gsa)
        drain_out(osa)
        transpose_chunk(ia, oa)
        fire_out(2 * N_PAIRS, oa, osa)
        drain_out(osb)

        @pl.when(wid == 0)
        def _():
            cp = pltpu.async_copy(
                wt_hbm.at[:, pl.ds((N_FULL - 1) * CW, CW)], ib, gsb)
            cp.wait()
            transpose_chunk(ib, ob)
            cpo = pltpu.async_copy(
                ob, out_hbm.at[pl.ds((N_FULL - 1) * (CW // 4), CW // 4)],
                osb)
            cpo.wait()
            cpt = pltpu.async_copy(tail_hbm, ob.at[pl.ds(0, TAIL // 4)], gsb)
            cpt.wait()
            cpt2 = pltpu.async_copy(
                ob.at[pl.ds(0, TAIL // 4)],
                out_hbm.at[pl.ds(N_FULL * (CW // 4), TAIL // 4)], osa)
            cpt2.wait()

        drain_out(osa)

    return body(WT, tail16)


def _lookup(tok1d, T128, n_t, n_s):
    n_sblk = n_s // SB
    t_half = n_t // 2

    @functools.partial(
        pl.kernel,
        out_type=jax.ShapeDtypeStruct((n_t, D_EMB, n_s), jnp.float32),
        mesh=_mesh(),
        scratch_types=(
            [pltpu.VMEM((SB,), jnp.int32) for _ in range(4)]
            + [pltpu.VMEM((SB, 128), jnp.float32) for _ in range(2)]
            + [pltpu.VMEM((D_EMB, SB), jnp.float32) for _ in range(2)]
            + [pltpu.SemaphoreType.DMA for _ in range(4)]
        ),
        compiler_params=pltpu.CompilerParams(use_tc_tiling_on_sc=True, needs_layout_passes=False),
    )
    def body(tok_hbm, w_hbm, out_hbm,
             iva, ivb, ova, ovb, ga, gb, oba, obb,
             gsa, gsb, osa, osb):
        wid = lax.axis_index("s") * NC + lax.axis_index("c")
        sblk = lax.rem(wid, n_sblk)
        t0 = lax.div(wid, n_sblk) * t_half
        s0 = sblk * SB
        iota = lax.iota(jnp.int32, 16)

        def stage_idx(t, iv, ov, gbuf, gsem):
            pltpu.sync_copy(tok_hbm.at[pl.ds(t * n_s + s0, SB)], iv)

            @pl.loop(0, SB // 16)
            def seg(i):
                v = iv[pl.ds(16 * i, 16)]
                ov[pl.ds(16 * i, 16)] = (v & 3) * D_EMB
                iv[pl.ds(16 * i, 16)] = lax.shift_right_logical(v, 2)
            for j in range(KG):
                pltpu.async_copy(
                    w_hbm.at[iv.at[pl.ds(128 * j, 128)]],
                    gbuf.at[pl.ds(128 * j, 128)], gsem)

        def drain_g(gbuf, gsem):
            for j in range(KG):
                pltpu.make_async_copy(
                    w_hbm.at[pl.ds(0, 128)],
                    gbuf.at[pl.ds(128 * j, 128)], gsem).wait()

        def extract(gbuf, ov, obuf):
            @plsc.parallel_loop(0, SB // 16, unroll=2)
            def seg(i):
                srow = 16 * i + iota
                ocol = ov[pl.ds(16 * i, 16)]
                for e in range(D_EMB):
                    obuf[e, pl.ds(16 * i, 16)] = plsc.load_gather(
                        gbuf, [srow, ocol + e])

        def fire_out(t, obuf, osem):
            return pltpu.async_copy(
                obuf, out_hbm.at[t, :, pl.ds(s0, SB)], osem)

        def drain_out(obuf, osem):
            pltpu.make_async_copy(
                obuf, out_hbm.at[0, :, pl.ds(s0, SB)], osem).wait()

        stage_idx(t0, iva, ova, ga, gsa)

        @pl.loop(0, t_half // 2)
        def pair(p):
            ta = t0 + 2 * p
            stage_idx(ta + 1, ivb, ovb, gb, gsb)
            drain_g(ga, gsa)

            @pl.when(p > 0)
            def _():
                drain_out(oba, osa)
            extract(ga, ova, oba)
            fire_out(ta, oba, osa)

            @pl.when(p + 1 < t_half // 2)
            def _():
                stage_idx(ta + 2, iva, ova, ga, gsa)
            drain_g(gb, gsb)

            @pl.when(p > 0)
            def _():
                drain_out(obb, osb)
            extract(gb, ovb, obb)
            fire_out(ta + 1, obb, osb)

        drain_out(oba, osa)
        drain_out(obb, osb)

    return body(tok1d, T128)


def kernel(tokens, W):
    n_seq, n_tok = tokens.shape
    WT = jnp.transpose(W)
    tail16 = (W[N_FULL * CW:] * SCALE).reshape(TAIL // 4, 128)
    T128 = _relayout(WT, tail16)
    tok1d = jnp.transpose(tokens).reshape(-1).astype(jnp.int32)
    out_phys = _lookup(tok1d, T128, n_tok, n_seq)
    return jnp.transpose(out_phys, (2, 0, 1))
